# baseline (speedup 1.0000x reference)
import dataclasses
import functools

import jax
import jax.numpy as jnp
from jax import lax
from jax.experimental import pallas as pl
from jax.experimental.pallas import tpu as pltpu
from jax.experimental.pallas import tpu_sc as plsc

N = 10000
D = 128
NPAD = 10240
TRASH = 10016
NSC = 2
NSUB = 16
NTILES = NSC * NSUB
EDGE_BLK = 128
BLKS_PER_TILE = 80
IDX_CHUNK = 8
NCHUNK = BLKS_PER_TILE // IDX_CHUNK
EPAD = NTILES * BLKS_PER_TILE * EDGE_BLK
STRIPE = NPAD // NSUB
DEGW = 16

_mesh = plsc.VectorSubcoreMesh(core_axis_name="c", subcore_axis_name="s")

_sc_params = pltpu.CompilerParams()
if "needs_layout_passes" in pltpu.CompilerParams.__dataclass_fields__:
    _sc_params = dataclasses.replace(_sc_params, needs_layout_passes=False)


@functools.partial(
    pl.kernel,
    out_type=jax.ShapeDtypeStruct((NTILES, NPAD), jnp.float32),
    mesh=_mesh,
    scratch_types=[
        pltpu.VMEM((BLKS_PER_TILE, EDGE_BLK), jnp.int32),
        pltpu.VMEM((NPAD,), jnp.float32),
    ],
    compiler_params=_sc_params,
)
def _sc_deg(src_hbm, zd_hbm, deg_out, src_i, deg_v):
    cid = lax.axis_index("c")
    sid = lax.axis_index("s")
    tile = cid * NSUB + sid
    pltpu.sync_copy(zd_hbm, deg_v)
    base = tile * BLKS_PER_TILE
    pltpu.sync_copy(src_hbm.at[pl.ds(base, BLKS_PER_TILE)], src_i)
    ones = jnp.full((16,), 1.0, jnp.float32)

    @pl.loop(0, BLKS_PER_TILE)
    def _(j):
        @pl.loop(0, EDGE_BLK // 16)
        def _(l):
            idx = src_i[j, pl.ds(l * 16, 16)]
            plsc.addupdate_scatter(deg_v, [idx], ones)

    pltpu.sync_copy(deg_v, deg_out.at[tile])


@functools.partial(
    pl.kernel,
    out_type=jax.ShapeDtypeStruct((NSC, NPAD, D), jnp.float32),
    mesh=_mesh,
    scratch_types=[
        pltpu.VMEM((IDX_CHUNK, EDGE_BLK), jnp.int32),
        pltpu.VMEM((IDX_CHUNK, EDGE_BLK), jnp.int32),
        pltpu.VMEM((IDX_CHUNK, EDGE_BLK), jnp.int32),
        pltpu.VMEM((IDX_CHUNK, EDGE_BLK), jnp.int32),
        pltpu.VMEM((EDGE_BLK, D), jnp.float32),
        pltpu.VMEM((EDGE_BLK, D), jnp.float32),
        pltpu.VMEM_SHARED((NPAD, D), jnp.float32),
        pltpu.SemaphoreType.DMA,
        pltpu.SemaphoreType.DMA,
        pltpu.SemaphoreType.DMA,
        pltpu.SemaphoreType.DMA,
        pltpu.SemaphoreType.DMA,
        pltpu.SemaphoreType.DMA,
    ],
)
def _sc_agg(h_hbm, src_hbm, dst_hbm, z_hbm,
            agg_out, dst_c0, dst_c1, src_c0, src_c1, rows_a, rows_b,
            agg_sh, sem_a, sem_b, sem_sa, sem_sb, sem_di, sem_si):
    cid = lax.axis_index("c")
    sid = lax.axis_index("s")
    tile = cid * NSUB + sid
    stripe = sid * STRIPE
    pltpu.sync_copy(z_hbm, agg_sh.at[pl.ds(stripe, STRIPE)])
    base = tile * BLKS_PER_TILE
    pltpu.sync_copy(dst_hbm.at[pl.ds(base, IDX_CHUNK)], dst_c0)
    pltpu.sync_copy(src_hbm.at[pl.ds(base, IDX_CHUNK)], src_c0)
    plsc.subcore_barrier()

    def do_chunk(c, dst_c, src_c, dst_n, src_n):
        @pl.when(c + 1 < NCHUNK)
        def _():
            off = base + (c + 1) * IDX_CHUNK
            pltpu.make_async_copy(dst_hbm.at[pl.ds(off, IDX_CHUNK)], dst_n,
                                  sem_di).start()
            pltpu.make_async_copy(src_hbm.at[pl.ds(off, IDX_CHUNK)], src_n,
                                  sem_si).start()

        pltpu.make_async_copy(h_hbm.at[dst_c.at[0]], rows_a, sem_a).start()

        @pl.loop(0, IDX_CHUNK, step=2)
        def _(j):
            pltpu.make_async_copy(h_hbm.at[dst_c.at[j]], rows_a, sem_a).wait()
            pltpu.async_copy(rows_a, agg_sh.at[src_c.at[j]],
                             sem_sa, add=True)

            @pl.when(j >= 1)
            def _():
                pltpu.make_async_copy(rows_b, agg_sh.at[src_c.at[j - 1]],
                                      sem_sb).wait()

            pltpu.make_async_copy(h_hbm.at[dst_c.at[j + 1]], rows_b,
                                  sem_b).start()
            pltpu.make_async_copy(h_hbm.at[dst_c.at[j + 1]], rows_b,
                                  sem_b).wait()
            pltpu.async_copy(rows_b, agg_sh.at[src_c.at[j + 1]],
                             sem_sb, add=True)
            pltpu.make_async_copy(rows_a, agg_sh.at[src_c.at[j]],
                                  sem_sa).wait()

            @pl.when(j + 2 < IDX_CHUNK)
            def _():
                pltpu.make_async_copy(h_hbm.at[dst_c.at[j + 2]], rows_a,
                                      sem_a).start()

        pltpu.make_async_copy(rows_b, agg_sh.at[src_c.at[IDX_CHUNK - 1]],
                              sem_sb).wait()

        @pl.when(c + 1 < NCHUNK)
        def _():
            pltpu.make_async_copy(dst_hbm.at[pl.ds(base, IDX_CHUNK)], dst_n,
                                  sem_di).wait()
            pltpu.make_async_copy(src_hbm.at[pl.ds(base, IDX_CHUNK)], src_n,
                                  sem_si).wait()

    @pl.loop(0, NCHUNK, step=2)
    def _(c):
        do_chunk(c, dst_c0, src_c0, dst_c1, src_c1)
        do_chunk(c + 1, dst_c1, src_c1, dst_c0, src_c0)

    plsc.subcore_barrier()
    pltpu.sync_copy(agg_sh.at[pl.ds(stripe, STRIPE)],
                    agg_out.at[cid].at[pl.ds(stripe, STRIPE)])


_ROWBLK = 512


def _proj(x, W):
    def body(x_r, w_r, o_r):
        o_r[...] = jnp.dot(x_r[...], w_r[...],
                           preferred_element_type=jnp.float32)

    return pl.pallas_call(
        body,
        grid=(NPAD // _ROWBLK,),
        in_specs=[
            pl.BlockSpec((_ROWBLK, D), lambda i: (i, 0)),
            pl.BlockSpec((D, D), lambda i: (0, 0)),
        ],
        out_specs=pl.BlockSpec((_ROWBLK, D), lambda i: (i, 0)),
        out_shape=jax.ShapeDtypeStruct((NPAD, D), jnp.float32),
    )(x, W)


def _combine(parts, degs, h, W, W2=None):
    fuse = W2 is not None

    def body(*refs):
        if fuse:
            p_r, d_r, h_r, w_r, w2_r, o_r = refs
        else:
            p_r, d_r, h_r, w_r, o_r = refs
        deg = jnp.sum(d_r[...], axis=1, keepdims=True)
        inv = 1.0 / jnp.maximum(deg, 1.0)
        agg = (p_r[0] + p_r[1]) * inv
        z = jnp.maximum(
            jnp.dot(agg + h_r[...], w_r[...],
                    preferred_element_type=jnp.float32), 0.0)
        if fuse:
            z = jnp.dot(z, w2_r[...], preferred_element_type=jnp.float32)
        o_r[...] = z

    in_specs = [
        pl.BlockSpec((NSC, _ROWBLK, D), lambda i: (0, i, 0)),
        pl.BlockSpec((_ROWBLK, NTILES), lambda i: (i, 0)),
        pl.BlockSpec((_ROWBLK, D), lambda i: (i, 0)),
        pl.BlockSpec((D, D), lambda i: (0, 0)),
    ]
    args = [parts, degs, h, W]
    if fuse:
        in_specs.append(pl.BlockSpec((D, D), lambda i: (0, 0)))
        args.append(W2)
    return pl.pallas_call(
        body,
        grid=(NPAD // _ROWBLK,),
        in_specs=in_specs,
        out_specs=pl.BlockSpec((_ROWBLK, D), lambda i: (i, 0)),
        out_shape=jax.ShapeDtypeStruct((NPAD, D), jnp.float32),
    )(*args)


def kernel(x, edge_index, W_in, W_layers, W_out):
    src = edge_index[0]
    dst = edge_index[1]
    e = src.shape[0]
    pad_e = EPAD - e
    pad_idx = N + (jnp.arange(pad_e, dtype=jnp.int32) % (NPAD - N))
    src_p = jnp.concatenate([src, pad_idx]).reshape(-1, EDGE_BLK)
    dst_p = jnp.concatenate([dst, pad_idx]).reshape(-1, EDGE_BLK)
    x_p = jnp.concatenate(
        [x, jnp.zeros((NPAD - N, D), jnp.float32)], axis=0)
    zeros_agg = jnp.zeros((STRIPE, D), jnp.float32)
    zeros_deg = jnp.zeros((NPAD,), jnp.float32)

    h = _proj(x_p, W_in)
    degs = _sc_deg(src_p, zeros_deg)
    degs_t = degs.T
    parts = _sc_agg(h, src_p, dst_p, zeros_agg)
    h = _combine(parts, degs_t, h, W_layers[0])
    parts2 = _sc_agg(h, src_p, dst_p, zeros_agg)
    out = _combine(parts2, degs_t, h, W_layers[1], W_out)
    return out[:N]


# speedup vs baseline: 1.0003x; 1.0003x over previous
import dataclasses
import functools

import jax
import jax.numpy as jnp
from jax import lax
from jax.experimental import pallas as pl
from jax.experimental.pallas import tpu as pltpu
from jax.experimental.pallas import tpu_sc as plsc

N = 10000
D = 128
NPAD = 10240
TRASH = 10016
NSC = 2
NSUB = 16
NTILES = NSC * NSUB
EDGE_BLK = 128
BLKS_PER_TILE = 80
IDX_CHUNK = 8
NCHUNK = BLKS_PER_TILE // IDX_CHUNK
EPAD = NTILES * BLKS_PER_TILE * EDGE_BLK
STRIPE = NPAD // NSUB
DEGW = 16

_mesh = plsc.VectorSubcoreMesh(core_axis_name="c", subcore_axis_name="s")

_sc_params = pltpu.CompilerParams()
if "needs_layout_passes" in pltpu.CompilerParams.__dataclass_fields__:
    _sc_params = dataclasses.replace(_sc_params, needs_layout_passes=False)


@functools.partial(
    pl.kernel,
    out_type=jax.ShapeDtypeStruct((NTILES, NPAD), jnp.float32),
    mesh=_mesh,
    scratch_types=[
        pltpu.VMEM((BLKS_PER_TILE, EDGE_BLK), jnp.int32),
        pltpu.VMEM((NPAD,), jnp.float32),
    ],
    compiler_params=_sc_params,
)
def _sc_deg(src_hbm, zd_hbm, deg_out, src_i, deg_v):
    cid = lax.axis_index("c")
    sid = lax.axis_index("s")
    tile = cid * NSUB + sid
    pltpu.sync_copy(zd_hbm, deg_v)
    base = tile * BLKS_PER_TILE
    pltpu.sync_copy(src_hbm.at[pl.ds(base, BLKS_PER_TILE)], src_i)
    ones = jnp.full((16,), 1.0, jnp.float32)

    @pl.loop(0, BLKS_PER_TILE)
    def _(j):
        @pl.loop(0, EDGE_BLK // 16)
        def _(l):
            idx = src_i[j, pl.ds(l * 16, 16)]
            plsc.addupdate_scatter(deg_v, [idx], ones)

    pltpu.sync_copy(deg_v, deg_out.at[tile])


@functools.partial(
    pl.kernel,
    out_type=jax.ShapeDtypeStruct((NSC, NPAD, D), jnp.float32),
    mesh=_mesh,
    scratch_types=[
        pltpu.VMEM((IDX_CHUNK, EDGE_BLK), jnp.int32),
        pltpu.VMEM((IDX_CHUNK, EDGE_BLK), jnp.int32),
        pltpu.VMEM((IDX_CHUNK, EDGE_BLK), jnp.int32),
        pltpu.VMEM((IDX_CHUNK, EDGE_BLK), jnp.int32),
        pltpu.VMEM((EDGE_BLK, D), jnp.float32),
        pltpu.VMEM((EDGE_BLK, D), jnp.float32),
        pltpu.VMEM_SHARED((NPAD, D), jnp.float32),
        pltpu.SemaphoreType.DMA,
        pltpu.SemaphoreType.DMA,
        pltpu.SemaphoreType.DMA,
        pltpu.SemaphoreType.DMA,
        pltpu.SemaphoreType.DMA,
        pltpu.SemaphoreType.DMA,
    ],
)
def _sc_agg(h_hbm, src_hbm, dst_hbm, z_hbm,
            agg_out, dst_c0, dst_c1, src_c0, src_c1, rows_a, rows_b,
            agg_sh, sem_a, sem_b, sem_sa, sem_sb, sem_di, sem_si):
    cid = lax.axis_index("c")
    sid = lax.axis_index("s")
    tile = cid * NSUB + sid
    stripe = sid * STRIPE
    pltpu.sync_copy(z_hbm, agg_sh.at[pl.ds(stripe, STRIPE)])
    base = tile * BLKS_PER_TILE
    pltpu.sync_copy(dst_hbm.at[pl.ds(base, IDX_CHUNK)], dst_c0)
    pltpu.sync_copy(src_hbm.at[pl.ds(base, IDX_CHUNK)], src_c0)
    plsc.subcore_barrier()

    def do_chunk(c, dst_c, src_c, dst_n, src_n):
        @pl.when(c + 1 < NCHUNK)
        def _():
            off = base + (c + 1) * IDX_CHUNK
            pltpu.make_async_copy(dst_hbm.at[pl.ds(off, IDX_CHUNK)], dst_n,
                                  sem_di).start()
            pltpu.make_async_copy(src_hbm.at[pl.ds(off, IDX_CHUNK)], src_n,
                                  sem_si).start()

        pltpu.make_async_copy(h_hbm.at[dst_c.at[0]], rows_a, sem_a).start()

        @pl.loop(0, IDX_CHUNK, step=2)
        def _(j):
            pltpu.make_async_copy(h_hbm.at[dst_c.at[j]], rows_a, sem_a).wait()
            pltpu.async_copy(rows_a, agg_sh.at[src_c.at[j]],
                             sem_sa, add=True)

            @pl.when(j >= 1)
            def _():
                pltpu.make_async_copy(rows_b, agg_sh.at[src_c.at[j - 1]],
                                      sem_sb).wait()

            pltpu.make_async_copy(h_hbm.at[dst_c.at[j + 1]], rows_b,
                                  sem_b).start()
            pltpu.make_async_copy(h_hbm.at[dst_c.at[j + 1]], rows_b,
                                  sem_b).wait()
            pltpu.async_copy(rows_b, agg_sh.at[src_c.at[j + 1]],
                             sem_sb, add=True)
            pltpu.make_async_copy(rows_a, agg_sh.at[src_c.at[j]],
                                  sem_sa).wait()

            @pl.when(j + 2 < IDX_CHUNK)
            def _():
                pltpu.make_async_copy(h_hbm.at[dst_c.at[j + 2]], rows_a,
                                      sem_a).start()

        pltpu.make_async_copy(rows_b, agg_sh.at[src_c.at[IDX_CHUNK - 1]],
                              sem_sb).wait()

        @pl.when(c + 1 < NCHUNK)
        def _():
            pltpu.make_async_copy(dst_hbm.at[pl.ds(base, IDX_CHUNK)], dst_n,
                                  sem_di).wait()
            pltpu.make_async_copy(src_hbm.at[pl.ds(base, IDX_CHUNK)], src_n,
                                  sem_si).wait()

    @pl.loop(0, NCHUNK, step=2)
    def _(c):
        do_chunk(c, dst_c0, src_c0, dst_c1, src_c1)
        do_chunk(c + 1, dst_c1, src_c1, dst_c0, src_c0)

    plsc.subcore_barrier()
    pltpu.sync_copy(agg_sh.at[pl.ds(stripe, STRIPE)],
                    agg_out.at[cid].at[pl.ds(stripe, STRIPE)])


_ROWBLK = 512


def _proj(x, W):
    def body(x_r, w_r, o_r):
        o_r[...] = jnp.dot(x_r[...], w_r[...],
                           preferred_element_type=jnp.float32)

    return pl.pallas_call(
        body,
        grid=(NPAD // _ROWBLK,),
        in_specs=[
            pl.BlockSpec((_ROWBLK, D), lambda i: (i, 0)),
            pl.BlockSpec((D, D), lambda i: (0, 0)),
        ],
        out_specs=pl.BlockSpec((_ROWBLK, D), lambda i: (i, 0)),
        out_shape=jax.ShapeDtypeStruct((NPAD, D), jnp.float32),
    )(x, W)


def _combine(parts, degs, h, W, W2=None):
    fuse = W2 is not None

    def body(*refs):
        if fuse:
            p_r, d_r, h_r, w_r, w2_r, o_r = refs
        else:
            p_r, d_r, h_r, w_r, o_r = refs
        deg = jnp.sum(d_r[...], axis=1, keepdims=True)
        inv = 1.0 / jnp.maximum(deg, 1.0)
        agg = (p_r[0] + p_r[1]) * inv
        z = jnp.maximum(
            jnp.dot(agg + h_r[...], w_r[...],
                    preferred_element_type=jnp.float32), 0.0)
        if fuse:
            z = jnp.dot(z, w2_r[...], preferred_element_type=jnp.float32)
        o_r[...] = z

    in_specs = [
        pl.BlockSpec((NSC, _ROWBLK, D), lambda i: (0, i, 0)),
        pl.BlockSpec((_ROWBLK, NTILES), lambda i: (i, 0)),
        pl.BlockSpec((_ROWBLK, D), lambda i: (i, 0)),
        pl.BlockSpec((D, D), lambda i: (0, 0)),
    ]
    args = [parts, degs, h, W]
    if fuse:
        in_specs.append(pl.BlockSpec((D, D), lambda i: (0, 0)))
        args.append(W2)
        out_specs = pl.BlockSpec((_ROWBLK, D), lambda i: (i, 0))
        out_shape = jax.ShapeDtypeStruct((NPAD, D), jnp.float32)
    else:
        out_specs = pl.BlockSpec((_ROWBLK, D), lambda i: (i, 0))
        out_shape = jax.ShapeDtypeStruct((NPAD, D), jnp.float32)
    return pl.pallas_call(
        body,
        grid=(NPAD // _ROWBLK,),
        in_specs=in_specs,
        out_specs=out_specs,
        out_shape=out_shape,
    )(*args)


def kernel(x, edge_index, W_in, W_layers, W_out):
    src = edge_index[0]
    dst = edge_index[1]
    e = src.shape[0]
    pad_e = EPAD - e
    pad_idx = N + (jnp.arange(pad_e, dtype=jnp.int32) % (NPAD - N))
    src_p = jnp.concatenate([src, pad_idx]).reshape(-1, EDGE_BLK)
    dst_p = jnp.concatenate([dst, pad_idx]).reshape(-1, EDGE_BLK)
    x_p = jnp.concatenate(
        [x, jnp.zeros((NPAD - N, D), jnp.float32)], axis=0)
    zeros_agg = jnp.zeros((STRIPE, D), jnp.float32)
    zeros_deg = jnp.zeros((NPAD,), jnp.float32)

    h = _proj(x_p, W_in)
    degs = _sc_deg(src_p, zeros_deg)
    degs_t = degs.T
    parts = _sc_agg(h, src_p, dst_p, zeros_agg)
    h = _combine(parts, degs_t, h, W_layers[0])
    parts2 = _sc_agg(h, src_p, dst_p, zeros_agg)
    out = _combine(parts2, degs_t, h, W_layers[1], W_out)
    return out[:N]


# speedup vs baseline: 1.0061x; 1.0058x over previous
import dataclasses
import functools

import jax
import jax.numpy as jnp
from jax import lax
from jax.experimental import pallas as pl
from jax.experimental.pallas import tpu as pltpu
from jax.experimental.pallas import tpu_sc as plsc

N = 10000
D = 128
NPAD = 10240
TRASH = 10016
NSC = 2
NSUB = 16
NTILES = NSC * NSUB
EDGE_BLK = 128
BLKS_PER_TILE = 80
IDX_CHUNK = 16
NCHUNK = BLKS_PER_TILE // IDX_CHUNK
EPAD = NTILES * BLKS_PER_TILE * EDGE_BLK
STRIPE = NPAD // NSUB
DEGW = 16

_mesh = plsc.VectorSubcoreMesh(core_axis_name="c", subcore_axis_name="s")

_sc_params = pltpu.CompilerParams()
if "needs_layout_passes" in pltpu.CompilerParams.__dataclass_fields__:
    _sc_params = dataclasses.replace(_sc_params, needs_layout_passes=False)


@functools.partial(
    pl.kernel,
    out_type=jax.ShapeDtypeStruct((NTILES, NPAD), jnp.float32),
    mesh=_mesh,
    scratch_types=[
        pltpu.VMEM((BLKS_PER_TILE, EDGE_BLK), jnp.int32),
        pltpu.VMEM((NPAD,), jnp.float32),
    ],
    compiler_params=_sc_params,
)
def _sc_deg(src_hbm, zd_hbm, deg_out, src_i, deg_v):
    cid = lax.axis_index("c")
    sid = lax.axis_index("s")
    tile = cid * NSUB + sid
    pltpu.sync_copy(zd_hbm, deg_v)
    base = tile * BLKS_PER_TILE
    pltpu.sync_copy(src_hbm.at[pl.ds(base, BLKS_PER_TILE)], src_i)
    ones = jnp.full((16,), 1.0, jnp.float32)

    @pl.loop(0, BLKS_PER_TILE)
    def _(j):
        @pl.loop(0, EDGE_BLK // 16)
        def _(l):
            idx = src_i[j, pl.ds(l * 16, 16)]
            plsc.addupdate_scatter(deg_v, [idx], ones)

    pltpu.sync_copy(deg_v, deg_out.at[tile])


@functools.partial(
    pl.kernel,
    out_type=jax.ShapeDtypeStruct((NSC, NPAD, D), jnp.float32),
    mesh=_mesh,
    scratch_types=[
        pltpu.VMEM((IDX_CHUNK, EDGE_BLK), jnp.int32),
        pltpu.VMEM((IDX_CHUNK, EDGE_BLK), jnp.int32),
        pltpu.VMEM((IDX_CHUNK, EDGE_BLK), jnp.int32),
        pltpu.VMEM((IDX_CHUNK, EDGE_BLK), jnp.int32),
        pltpu.VMEM((EDGE_BLK, D), jnp.float32),
        pltpu.VMEM((EDGE_BLK, D), jnp.float32),
        pltpu.VMEM_SHARED((NPAD, D), jnp.float32),
        pltpu.SemaphoreType.DMA,
        pltpu.SemaphoreType.DMA,
        pltpu.SemaphoreType.DMA,
        pltpu.SemaphoreType.DMA,
        pltpu.SemaphoreType.DMA,
        pltpu.SemaphoreType.DMA,
    ],
)
def _sc_agg(h_hbm, src_hbm, dst_hbm, z_hbm,
            agg_out, dst_c0, dst_c1, src_c0, src_c1, rows_a, rows_b,
            agg_sh, sem_a, sem_b, sem_sa, sem_sb, sem_di, sem_si):
    cid = lax.axis_index("c")
    sid = lax.axis_index("s")
    tile = cid * NSUB + sid
    stripe = sid * STRIPE
    pltpu.sync_copy(z_hbm, agg_sh.at[pl.ds(stripe, STRIPE)])
    base = tile * BLKS_PER_TILE
    pltpu.sync_copy(dst_hbm.at[pl.ds(base, IDX_CHUNK)], dst_c0)
    pltpu.sync_copy(src_hbm.at[pl.ds(base, IDX_CHUNK)], src_c0)
    plsc.subcore_barrier()

    def do_chunk(c, dst_c, src_c, dst_n, src_n):
        if c + 1 < NCHUNK:
            off = base + (c + 1) * IDX_CHUNK
            pltpu.make_async_copy(dst_hbm.at[pl.ds(off, IDX_CHUNK)], dst_n,
                                  sem_di).start()
            pltpu.make_async_copy(src_hbm.at[pl.ds(off, IDX_CHUNK)], src_n,
                                  sem_si).start()


        @pl.loop(0, IDX_CHUNK, step=2)
        def _(j):
            pltpu.make_async_copy(h_hbm.at[dst_c.at[j]], rows_a, sem_a).wait()
            pltpu.async_copy(rows_a, agg_sh.at[src_c.at[j]],
                             sem_sa, add=True)

            @pl.when(j >= 1)
            def _():
                pltpu.make_async_copy(rows_b, agg_sh.at[src_c.at[j - 1]],
                                      sem_sb).wait()

            pltpu.make_async_copy(h_hbm.at[dst_c.at[j + 1]], rows_b,
                                  sem_b).start()
            pltpu.make_async_copy(h_hbm.at[dst_c.at[j + 1]], rows_b,
                                  sem_b).wait()
            pltpu.async_copy(rows_b, agg_sh.at[src_c.at[j + 1]],
                             sem_sb, add=True)
            pltpu.make_async_copy(rows_a, agg_sh.at[src_c.at[j]],
                                  sem_sa).wait()

            @pl.when(j + 2 < IDX_CHUNK)
            def _():
                pltpu.make_async_copy(h_hbm.at[dst_c.at[j + 2]], rows_a,
                                      sem_a).start()

        pltpu.make_async_copy(rows_b, agg_sh.at[src_c.at[IDX_CHUNK - 1]],
                              sem_sb).wait()

        if c + 1 < NCHUNK:
            pltpu.make_async_copy(dst_hbm.at[pl.ds(base, IDX_CHUNK)], dst_n,
                                  sem_di).wait()
            pltpu.make_async_copy(src_hbm.at[pl.ds(base, IDX_CHUNK)], src_n,
                                  sem_si).wait()
            pltpu.make_async_copy(h_hbm.at[dst_n.at[0]], rows_a,
                                  sem_a).start()

    bufs = [(dst_c0, src_c0, dst_c1, src_c1), (dst_c1, src_c1, dst_c0, src_c0)]
    pltpu.make_async_copy(h_hbm.at[dst_c0.at[0]], rows_a, sem_a).start()
    for c in range(NCHUNK):
        do_chunk(c, *bufs[c % 2])

    plsc.subcore_barrier()
    pltpu.sync_copy(agg_sh.at[pl.ds(stripe, STRIPE)],
                    agg_out.at[cid].at[pl.ds(stripe, STRIPE)])


_ROWBLK = 512


def _proj(x, W):
    def body(x_r, w_r, o_r):
        o_r[...] = jnp.dot(x_r[...], w_r[...],
                           preferred_element_type=jnp.float32)

    return pl.pallas_call(
        body,
        grid=(NPAD // _ROWBLK,),
        in_specs=[
            pl.BlockSpec((_ROWBLK, D), lambda i: (i, 0)),
            pl.BlockSpec((D, D), lambda i: (0, 0)),
        ],
        out_specs=pl.BlockSpec((_ROWBLK, D), lambda i: (i, 0)),
        out_shape=jax.ShapeDtypeStruct((NPAD, D), jnp.float32),
    )(x, W)


def _combine(parts, degs, h, W, W2=None):
    fuse = W2 is not None

    def body(*refs):
        if fuse:
            p_r, d_r, h_r, w_r, w2_r, o_r = refs
        else:
            p_r, d_r, h_r, w_r, o_r = refs
        deg = jnp.sum(d_r[...], axis=1, keepdims=True)
        inv = 1.0 / jnp.maximum(deg, 1.0)
        agg = (p_r[0] + p_r[1]) * inv
        z = jnp.maximum(
            jnp.dot(agg + h_r[...], w_r[...],
                    preferred_element_type=jnp.float32), 0.0)
        if fuse:
            z = jnp.dot(z, w2_r[...], preferred_element_type=jnp.float32)
        o_r[...] = z

    in_specs = [
        pl.BlockSpec((NSC, _ROWBLK, D), lambda i: (0, i, 0)),
        pl.BlockSpec((_ROWBLK, NTILES), lambda i: (i, 0)),
        pl.BlockSpec((_ROWBLK, D), lambda i: (i, 0)),
        pl.BlockSpec((D, D), lambda i: (0, 0)),
    ]
    args = [parts, degs, h, W]
    if fuse:
        in_specs.append(pl.BlockSpec((D, D), lambda i: (0, 0)))
        args.append(W2)
        out_specs = pl.BlockSpec((_ROWBLK, D), lambda i: (i, 0))
        out_shape = jax.ShapeDtypeStruct((NPAD, D), jnp.float32)
    else:
        out_specs = pl.BlockSpec((_ROWBLK, D), lambda i: (i, 0))
        out_shape = jax.ShapeDtypeStruct((NPAD, D), jnp.float32)
    return pl.pallas_call(
        body,
        grid=(NPAD // _ROWBLK,),
        in_specs=in_specs,
        out_specs=out_specs,
        out_shape=out_shape,
    )(*args)


def kernel(x, edge_index, W_in, W_layers, W_out):
    src = edge_index[0]
    dst = edge_index[1]
    e = src.shape[0]
    pad_e = EPAD - e
    pad_idx = N + (jnp.arange(pad_e, dtype=jnp.int32) % (NPAD - N))
    src_p = jnp.concatenate([src, pad_idx]).reshape(-1, EDGE_BLK)
    dst_p = jnp.concatenate([dst, pad_idx]).reshape(-1, EDGE_BLK)
    x_p = jnp.concatenate(
        [x, jnp.zeros((NPAD - N, D), jnp.float32)], axis=0)
    zeros_agg = jnp.zeros((STRIPE, D), jnp.float32)
    zeros_deg = jnp.zeros((NPAD,), jnp.float32)

    h = _proj(x_p, W_in)
    degs = _sc_deg(src_p, zeros_deg)
    degs_t = degs.T
    parts = _sc_agg(h, src_p, dst_p, zeros_agg)
    h = _combine(parts, degs_t, h, W_layers[0])
    parts2 = _sc_agg(h, src_p, dst_p, zeros_agg)
    out = _combine(parts2, degs_t, h, W_layers[1], W_out)
    return out[:N]


# speedup vs baseline: 1.0109x; 1.0048x over previous
import dataclasses
import functools

import jax
import jax.numpy as jnp
from jax import lax
from jax.experimental import pallas as pl
from jax.experimental.pallas import tpu as pltpu
from jax.experimental.pallas import tpu_sc as plsc

N = 10000
D = 128
NPAD = 10240
TRASH = 10016
NSC = 2
NSUB = 16
NTILES = NSC * NSUB
EDGE_BLK = 128
BLKS_PER_TILE = 80
IDX_CHUNK = 16
NCHUNK = BLKS_PER_TILE // IDX_CHUNK
EPAD = NTILES * BLKS_PER_TILE * EDGE_BLK
STRIPE = NPAD // NSUB
DEGW = 16

_mesh = plsc.VectorSubcoreMesh(core_axis_name="c", subcore_axis_name="s")

_sc_params = pltpu.CompilerParams()
if "needs_layout_passes" in pltpu.CompilerParams.__dataclass_fields__:
    _sc_params = dataclasses.replace(_sc_params, needs_layout_passes=False)


@functools.partial(
    pl.kernel,
    out_type=jax.ShapeDtypeStruct((NTILES, NPAD), jnp.float32),
    mesh=_mesh,
    scratch_types=[
        pltpu.VMEM((BLKS_PER_TILE, EDGE_BLK), jnp.int32),
        pltpu.VMEM((NPAD,), jnp.float32),
    ],
    compiler_params=_sc_params,
)
def _sc_deg(src_hbm, zd_hbm, deg_out, src_i, deg_v):
    cid = lax.axis_index("c")
    sid = lax.axis_index("s")
    tile = cid * NSUB + sid
    pltpu.sync_copy(zd_hbm, deg_v)
    base = tile * BLKS_PER_TILE
    pltpu.sync_copy(src_hbm.at[pl.ds(base, BLKS_PER_TILE)], src_i)
    ones = jnp.full((16,), 1.0, jnp.float32)

    @pl.loop(0, BLKS_PER_TILE)
    def _(j):
        @pl.loop(0, EDGE_BLK // 16)
        def _(l):
            idx = src_i[j, pl.ds(l * 16, 16)]
            plsc.addupdate_scatter(deg_v, [idx], ones)

    pltpu.sync_copy(deg_v, deg_out.at[tile])


@functools.partial(
    pl.kernel,
    out_type=jax.ShapeDtypeStruct((NSC, NPAD, D), jnp.float32),
    mesh=_mesh,
    scratch_types=[
        pltpu.VMEM((IDX_CHUNK, EDGE_BLK), jnp.int32),
        pltpu.VMEM((IDX_CHUNK, EDGE_BLK), jnp.int32),
        pltpu.VMEM((IDX_CHUNK, EDGE_BLK), jnp.int32),
        pltpu.VMEM((IDX_CHUNK, EDGE_BLK), jnp.int32),
        pltpu.VMEM((EDGE_BLK, D), jnp.float32),
        pltpu.VMEM((EDGE_BLK, D), jnp.float32),
        pltpu.VMEM_SHARED((NPAD, D), jnp.float32),
        pltpu.SemaphoreType.DMA,
        pltpu.SemaphoreType.DMA,
        pltpu.SemaphoreType.DMA,
        pltpu.SemaphoreType.DMA,
        pltpu.SemaphoreType.DMA,
        pltpu.SemaphoreType.DMA,
    ],
)
def _sc_agg(h_hbm, src_hbm, dst_hbm, z_hbm,
            agg_out, dst_c0, dst_c1, src_c0, src_c1, rows_a, rows_b,
            agg_sh, sem_a, sem_b, sem_sa, sem_sb, sem_di, sem_si):
    cid = lax.axis_index("c")
    sid = lax.axis_index("s")
    tile = cid * NSUB + sid
    stripe = sid * STRIPE
    pltpu.sync_copy(z_hbm, agg_sh.at[pl.ds(stripe, STRIPE)])
    base = tile * BLKS_PER_TILE
    pltpu.sync_copy(dst_hbm.at[pl.ds(base, IDX_CHUNK)], dst_c0)
    pltpu.sync_copy(src_hbm.at[pl.ds(base, IDX_CHUNK)], src_c0)
    plsc.subcore_barrier()

    def do_chunk(c, dst_c, src_c, dst_n, src_n):
        if c + 1 < NCHUNK:
            off = base + (c + 1) * IDX_CHUNK
            pltpu.make_async_copy(dst_hbm.at[pl.ds(off, IDX_CHUNK)], dst_n,
                                  sem_di).start()
            pltpu.make_async_copy(src_hbm.at[pl.ds(off, IDX_CHUNK)], src_n,
                                  sem_si).start()


        @pl.loop(0, IDX_CHUNK, step=2)
        def _(j):
            pltpu.make_async_copy(h_hbm.at[dst_c.at[j]], rows_a, sem_a).wait()
            pltpu.async_copy(rows_a, agg_sh.at[src_c.at[j]],
                             sem_sa, add=True)

            @pl.when(j >= 1)
            def _():
                pltpu.make_async_copy(rows_b, agg_sh.at[src_c.at[j - 1]],
                                      sem_sb).wait()

            pltpu.make_async_copy(h_hbm.at[dst_c.at[j + 1]], rows_b,
                                  sem_b).start()
            pltpu.make_async_copy(h_hbm.at[dst_c.at[j + 1]], rows_b,
                                  sem_b).wait()
            pltpu.async_copy(rows_b, agg_sh.at[src_c.at[j + 1]],
                             sem_sb, add=True)
            pltpu.make_async_copy(rows_a, agg_sh.at[src_c.at[j]],
                                  sem_sa).wait()

            @pl.when(j + 2 < IDX_CHUNK)
            def _():
                pltpu.make_async_copy(h_hbm.at[dst_c.at[j + 2]], rows_a,
                                      sem_a).start()

        pltpu.make_async_copy(rows_b, agg_sh.at[src_c.at[IDX_CHUNK - 1]],
                              sem_sb).wait()

        if c + 1 < NCHUNK:
            pltpu.make_async_copy(dst_hbm.at[pl.ds(base, IDX_CHUNK)], dst_n,
                                  sem_di).wait()
            pltpu.make_async_copy(src_hbm.at[pl.ds(base, IDX_CHUNK)], src_n,
                                  sem_si).wait()
            pltpu.make_async_copy(h_hbm.at[dst_n.at[0]], rows_a,
                                  sem_a).start()

    bufs = [(dst_c0, src_c0, dst_c1, src_c1), (dst_c1, src_c1, dst_c0, src_c0)]
    pltpu.make_async_copy(h_hbm.at[dst_c0.at[0]], rows_a, sem_a).start()
    for c in range(NCHUNK):
        do_chunk(c, *bufs[c % 2])

    plsc.subcore_barrier()
    pltpu.sync_copy(agg_sh.at[pl.ds(stripe, STRIPE)],
                    agg_out.at[cid].at[pl.ds(stripe, STRIPE)])


_ROWBLK = 512


def _proj(x, W):
    def body(x_r, w_r, o_r):
        o_r[...] = jnp.dot(x_r[...], w_r[...],
                           preferred_element_type=jnp.float32)

    return pl.pallas_call(
        body,
        grid=(NPAD // _ROWBLK,),
        in_specs=[
            pl.BlockSpec((_ROWBLK, D), lambda i: (i, 0)),
            pl.BlockSpec((D, D), lambda i: (0, 0)),
        ],
        out_specs=pl.BlockSpec((_ROWBLK, D), lambda i: (i, 0)),
        out_shape=jax.ShapeDtypeStruct((NPAD, D), jnp.float32),
    )(x, W)


def _combine(parts, degs, h, W, W2=None):
    fuse = W2 is not None

    def body(*refs):
        if fuse:
            p_r, d_r, h_r, w_r, w2_r, o_r = refs
        else:
            p_r, d_r, h_r, w_r, o_r = refs
        deg = jnp.sum(d_r[...], axis=1, keepdims=True)
        inv = 1.0 / jnp.maximum(deg, 1.0)
        agg = (p_r[0] + p_r[1]) * inv
        z = jnp.maximum(
            jnp.dot(agg + h_r[...], w_r[...],
                    preferred_element_type=jnp.float32), 0.0)
        if fuse:
            z = jnp.dot(z, w2_r[...], preferred_element_type=jnp.float32)
        o_r[...] = z

    in_specs = [
        pl.BlockSpec((NSC, _ROWBLK, D), lambda i: (0, i, 0)),
        pl.BlockSpec((_ROWBLK, NTILES), lambda i: (i, 0)),
        pl.BlockSpec((_ROWBLK, D), lambda i: (i, 0)),
        pl.BlockSpec((D, D), lambda i: (0, 0)),
    ]
    args = [parts, degs, h, W]
    if fuse:
        rb = 400
        in_specs = [
            pl.BlockSpec((NSC, rb, D), lambda i: (0, i, 0)),
            pl.BlockSpec((rb, NTILES), lambda i: (i, 0)),
            pl.BlockSpec((rb, D), lambda i: (i, 0)),
            pl.BlockSpec((D, D), lambda i: (0, 0)),
            pl.BlockSpec((D, D), lambda i: (0, 0)),
        ]
        args = [parts, degs, h, W, W2]
        grid = (N // rb,)
        out_specs = pl.BlockSpec((rb, D), lambda i: (i, 0))
        out_shape = jax.ShapeDtypeStruct((N, D), jnp.float32)
    else:
        grid = (NPAD // _ROWBLK,)
        out_specs = pl.BlockSpec((_ROWBLK, D), lambda i: (i, 0))
        out_shape = jax.ShapeDtypeStruct((NPAD, D), jnp.float32)
    return pl.pallas_call(
        body,
        grid=grid,
        in_specs=in_specs,
        out_specs=out_specs,
        out_shape=out_shape,
    )(*args)


def kernel(x, edge_index, W_in, W_layers, W_out):
    src = edge_index[0]
    dst = edge_index[1]
    e = src.shape[0]
    pad_e = EPAD - e
    pad_idx = N + (jnp.arange(pad_e, dtype=jnp.int32) % (NPAD - N))
    src_p = jnp.concatenate([src, pad_idx]).reshape(-1, EDGE_BLK)
    dst_p = jnp.concatenate([dst, pad_idx]).reshape(-1, EDGE_BLK)
    x_p = jnp.concatenate(
        [x, jnp.zeros((NPAD - N, D), jnp.float32)], axis=0)
    zeros_agg = jnp.zeros((STRIPE, D), jnp.float32)
    zeros_deg = jnp.zeros((NPAD,), jnp.float32)

    h = _proj(x_p, W_in)
    degs = _sc_deg(src_p, zeros_deg)
    degs_t = degs.T
    parts = _sc_agg(h, src_p, dst_p, zeros_agg)
    h = _combine(parts, degs_t, h, W_layers[0])
    parts2 = _sc_agg(h, src_p, dst_p, zeros_agg)
    return _combine(parts2, degs_t, h, W_layers[1], W_out)
